# baseline (speedup 1.0000x reference)
import functools

import jax
import jax.numpy as jnp
from jax import lax
from jax.experimental import pallas as pl
from jax.experimental.pallas import tpu as pltpu
from jax.experimental.pallas import tpu_sc as plsc

NUM_EMB = 100000
DIM = 64
BATCH = 4096 * 50

NUM_CORES = 2
NUM_SUBCORES = 16
NW = NUM_CORES * NUM_SUBCORES
B_PER_W = BATCH // NW
CHUNK = 400
NCHUNK = B_PER_W // CHUNK
NBUF = 4

_mesh = plsc.VectorSubcoreMesh(core_axis_name="c", subcore_axis_name="s")


@functools.partial(
    pl.kernel,
    mesh=_mesh,
    compiler_params=pltpu.CompilerParams(use_tc_tiling_on_sc=False),
    out_type=jax.ShapeDtypeStruct((BATCH, DIM), jnp.float32),
    scratch_types=[
        pltpu.VMEM((B_PER_W,), jnp.int32),
        [pltpu.VMEM((CHUNK, DIM), jnp.float32) for _ in range(NBUF)],
        [pltpu.SemaphoreType.DMA for _ in range(NBUF)],
        [pltpu.SemaphoreType.DMA for _ in range(NBUF)],
    ],
)
def _emb_lookup(ids_hbm, table_hbm, out_hbm, idx_v, bufs, gsems, wsems):
    wid = lax.axis_index("s") * NUM_CORES + lax.axis_index("c")
    base = wid * B_PER_W
    pltpu.sync_copy(ids_hbm.at[pl.ds(base, B_PER_W)], idx_v)

    ghandles = [None] * NBUF
    whandles = [None] * NBUF

    def start_gather(j):
        b = j % NBUF
        if whandles[b] is not None:
            whandles[b].wait()
        ghandles[b] = pltpu.async_copy(
            table_hbm.at[idx_v.at[pl.ds(j * CHUNK, CHUNK)]], bufs[b], gsems[b])

    for j in range(min(NBUF - 1, NCHUNK)):
        start_gather(j)

    for i in range(NCHUNK):
        b = i % NBUF
        j = i + NBUF - 1
        if j < NCHUNK:
            start_gather(j)
        ghandles[b].wait()
        whandles[b] = pltpu.async_copy(
            bufs[b], out_hbm.at[pl.ds(base + i * CHUNK, CHUNK)], wsems[b])
    for b in range(NBUF):
        if whandles[b] is not None:
            whandles[b].wait()


def kernel(token_ids, weight):
    one = jax.lax.optimization_barrier(jnp.float32(1.0))
    table2 = (jnp.pad(weight, ((0, 0), (0, 128 - DIM))) * one).reshape(
        2 * NUM_EMB, DIM)
    flat_ids = token_ids.reshape(-1).astype(jnp.int32) * 2
    out = _emb_lookup(flat_ids, table2)
    return (out * one).reshape(token_ids.shape + (DIM,))


# speedup vs baseline: 1.5606x; 1.5606x over previous
import functools

import jax
import jax.numpy as jnp
from jax import lax
from jax.experimental import pallas as pl
from jax.experimental.pallas import tpu as pltpu
from jax.experimental.pallas import tpu_sc as plsc

_COMPILER_PARAMS = pltpu.CompilerParams(use_tc_tiling_on_sc=False)

NUM_EMB = 100000
DIM = 64
PDIM = 128
NSEQ_TOT = 4096
SEQ = 50

NUM_CORES = 2
NUM_SUBCORES = 16
NW = NUM_CORES * NUM_SUBCORES
SEQ_PER_W = NSEQ_TOT // NW
IDS_PER_W = SEQ_PER_W * SEQ
SEQ_CHUNK = 8
ROWS = SEQ_CHUNK * SEQ
NCHUNK = SEQ_PER_W // SEQ_CHUNK

_mesh = plsc.VectorSubcoreMesh(core_axis_name="c", subcore_axis_name="s")


@functools.partial(
    pl.kernel,
    mesh=_mesh,
    compiler_params=_COMPILER_PARAMS,
    out_type=jax.ShapeDtypeStruct((NSEQ_TOT * SEQ, DIM), jnp.float32),
    scratch_types=[
        pltpu.VMEM((IDS_PER_W,), jnp.int32),
        [pltpu.VMEM((ROWS, PDIM), jnp.float32) for _ in range(2)],
        [pltpu.SemaphoreType.DMA for _ in range(2)],
        [pltpu.SemaphoreType.DMA for _ in range(2)],
    ],
)
def _emb_lookup(ids_hbm, table_hbm, out_hbm, idx_v, bufs, gsems, wsems):
    wid = lax.axis_index("s") * NUM_CORES + lax.axis_index("c")
    row_base = wid * IDS_PER_W
    pltpu.sync_copy(ids_hbm.at[pl.ds(row_base, IDS_PER_W)], idx_v)

    ghandles = [None, None]
    whandles = [None, None]

    def start_gather(j):
        b = j % 2
        if whandles[b] is not None:
            whandles[b].wait()
        ghandles[b] = pltpu.async_copy(
            table_hbm.at[idx_v.at[pl.ds(j * ROWS, ROWS)]], bufs[b], gsems[b])

    start_gather(0)
    for i in range(NCHUNK):
        b = i % 2
        if i + 1 < NCHUNK:
            start_gather(i + 1)
        ghandles[b].wait()
        whandles[b] = pltpu.async_copy(
            bufs[b].at[:, pl.ds(0, DIM)],
            out_hbm.at[pl.ds(row_base + i * ROWS, ROWS)], wsems[b])
    for h in whandles:
        if h is not None:
            h.wait()


def kernel(token_ids, weight):
    table2 = jnp.pad(weight, ((0, 0), (0, PDIM - DIM)))
    flat_ids = token_ids.reshape(-1).astype(jnp.int32)
    out = _emb_lookup(flat_ids, table2)
    return out.reshape(NSEQ_TOT, SEQ, DIM)
